# baseline (speedup 1.0000x reference)
import math

import jax
import jax.numpy as jnp
from jax import lax
from jax.experimental import pallas as pl
from jax.experimental.pallas import tpu as pltpu
from jax.experimental.pallas import tpu_sc as plsc

N_NODES = 10000
N_EVENTS = 320000
D = 128

NC = 2
NS = 16
L = 16
NW = NC * NS
PER_TILE = N_EVENTS // NW
CH = 64
NB = 2
N_FULL = PER_TILE // CH
CH_T = PER_TILE - N_FULL * CH

_COS_COEF = [0.9999999909693997, -19.739111734976838, 64.93226208899004,
             -85.30666410430229, 58.93072762676036, -21.268695769782525]
_NCOEF = len(_COS_COEF)
_BIG = 1.5 * 2.0 ** 23


def _body(mem_hbm, lu_hbm, src_hbm, dst_hbm, ts_hbm, feat_hbm, idx_hbm,
          tw_hbm, tb_hbm, out_hbm,
          lu_v, tw_v, tb_v, sid_c, did_c, eid_c, ts_c, dt_c, big,
          gsem, ixsem0, ixsem1, stsem0, stsem1):
    sub = lax.axis_index("s")
    wid = sub * NC + lax.axis_index("c")
    t0 = wid * PER_TILE

    ixsem = (ixsem0, ixsem1)
    stsem = (stsem0, stsem1)

    pltpu.sync_copy(lu_hbm, lu_v)
    pltpu.sync_copy(tw_hbm, tw_v)
    pltpu.sync_copy(tb_hbm, tb_v)

    ws = tuple(tw_v[pl.ds(16 * j, 16)] for j in range(8))
    bs = tuple(tb_v[pl.ds(16 * j, 16)] for j in range(8))

    def stage(c, b, n):
        lo = t0 + c * CH
        pltpu.async_copy(src_hbm.at[pl.ds(lo, n)],
                         sid_c.at[b, pl.ds(0, n)], ixsem[b])
        pltpu.async_copy(dst_hbm.at[pl.ds(lo, n)],
                         did_c.at[b, pl.ds(0, n)], ixsem[b])
        pltpu.async_copy(idx_hbm.at[pl.ds(lo, n)],
                         eid_c.at[b, pl.ds(0, n)], ixsem[b])
        pltpu.async_copy(ts_hbm.at[pl.ds(lo, n)],
                         ts_c.at[b, pl.ds(0, n)], ixsem[b])

    def wait_stage(b, n):
        for ref in (sid_c, did_c, eid_c):
            pltpu.make_async_copy(src_hbm.at[pl.ds(0, n)],
                                  ref.at[b, pl.ds(0, n)], ixsem[b]).wait()
        pltpu.make_async_copy(ts_hbm.at[pl.ds(0, n)],
                              ts_c.at[b, pl.ds(0, n)], ixsem[b]).wait()

    def drain_store(b, n):
        pltpu.make_async_copy(big.at[b, pl.ds(0, n)],
                              out_hbm.at[pl.ds(0, n)], stsem[b]).wait()

    def process(c, b, n, stage_next):
        base = t0 + c * CH
        nb = (b + 1) % NB

        wait_stage(b, n)
        d1 = pltpu.async_copy(mem_hbm.at[sid_c.at[b, pl.ds(0, n)]],
                              big.at[b, pl.ds(0, n), pl.ds(0, D)], gsem)
        d2 = pltpu.async_copy(mem_hbm.at[did_c.at[b, pl.ds(0, n)]],
                              big.at[b, pl.ds(0, n), pl.ds(D, D)], gsem)
        d3 = pltpu.async_copy(feat_hbm.at[eid_c.at[b, pl.ds(0, n)]],
                              big.at[b, pl.ds(0, n), pl.ds(3 * D, D)], gsem)
        if stage_next:
            @pl.when(c + 1 < N_FULL)
            def _():
                stage(c + 1, nb, CH)

            @pl.when(c + 1 == N_FULL)
            def _():
                stage(N_FULL, nb, CH_T)

        for i in range(n // L):
            idx16 = sid_c[b, pl.ds(i * L, L)]
            lu16 = plsc.load_gather(lu_v, [idx16])
            dt_c[b, pl.ds(i * L, L)] = ts_c[b, pl.ds(i * L, L)] - lu16

        def ev(e, carry):
            eidx = jnp.zeros((L,), jnp.int32) + e
            d = plsc.load_gather(dt_c.at[b], [eidx])
            for j in range(8):
                t = d * ws[j] + bs[j]
                nn = (t + _BIG) - _BIG
                r = t - nn
                s = r * r
                p = jnp.float32(_COS_COEF[_NCOEF - 1])
                for k in range(_NCOEF - 2, -1, -1):
                    p = p * s + jnp.float32(_COS_COEF[k])
                big[b, e, pl.ds(2 * D + 16 * j, 16)] = p
            return carry
        lax.fori_loop(0, n, ev, 0)

        d1.wait()
        d2.wait()
        d3.wait()

        pltpu.async_copy(big.at[b, pl.ds(0, n)],
                         out_hbm.at[pl.ds(base, n)], stsem[b])

    stage(0, 0, CH)

    def pair(i, carry):
        for b in range(NB):
            @pl.when(i >= 1)
            def _(b=b):
                drain_store(b, CH)
            process(NB * i + b, b, CH, stage_next=True)
        return carry

    lax.fori_loop(0, N_FULL // NB, pair, 0)
    drain_store(0, CH)
    process(jnp.int32(N_FULL), 0, CH_T, stage_next=False)

    drain_store(1, CH)
    drain_store(0, CH_T)


_sc_call = pl.kernel(
    _body,
    out_type=jax.ShapeDtypeStruct((N_EVENTS, 4 * D), jnp.float32),
    mesh=plsc.VectorSubcoreMesh(core_axis_name="c", subcore_axis_name="s",
                                num_cores=NC, num_subcores=NS),
    compiler_params=pltpu.CompilerParams(needs_layout_passes=False),
    scratch_types=[
        pltpu.VMEM((N_NODES,), jnp.float32),
        pltpu.VMEM((D,), jnp.float32),
        pltpu.VMEM((D,), jnp.float32),
        pltpu.VMEM((NB, CH), jnp.int32),
        pltpu.VMEM((NB, CH), jnp.int32),
        pltpu.VMEM((NB, CH), jnp.int32),
        pltpu.VMEM((NB, CH), jnp.float32),
        pltpu.VMEM((NB, CH), jnp.float32),
        pltpu.VMEM((NB, CH, 4 * D), jnp.float32),
        pltpu.SemaphoreType.DMA,
        pltpu.SemaphoreType.DMA,
        pltpu.SemaphoreType.DMA,
        pltpu.SemaphoreType.DMA,
        pltpu.SemaphoreType.DMA,
    ],
)

def kernel(memory, last_update, src_nodes, dst_nodes, timestamps,
           event_features, indices, te_w, te_b):
    inv2pi = jnp.float32(1.0 / (2.0 * math.pi))
    return _sc_call(
        memory,
        last_update,
        src_nodes.astype(jnp.int32),
        dst_nodes.astype(jnp.int32),
        timestamps,
        event_features,
        indices.astype(jnp.int32),
        (te_w * inv2pi).astype(jnp.float32),
        (te_b * inv2pi).astype(jnp.float32),
    )


# speedup vs baseline: 1.0035x; 1.0035x over previous
import math

import jax
import jax.numpy as jnp
from jax import lax
from jax.experimental import pallas as pl
from jax.experimental.pallas import tpu as pltpu
from jax.experimental.pallas import tpu_sc as plsc

N_NODES = 10000
N_EVENTS = 320000
D = 128

NC = 2
NS = 16
L = 16
NW = NC * NS
PER_TILE = N_EVENTS // NW
CH = 96
NB = 2
N_FULL = PER_TILE // CH
CH_T = PER_TILE - N_FULL * CH

_COS_COEF = [0.9999999909693997, -19.739111734976838, 64.93226208899004,
             -85.30666410430229, 58.93072762676036, -21.268695769782525]
_NCOEF = len(_COS_COEF)
_BIG = 1.5 * 2.0 ** 23


def _body(mem_hbm, lu_hbm, src_hbm, dst_hbm, ts_hbm, feat_hbm, idx_hbm,
          tw_hbm, tb_hbm, out_hbm,
          lu_v, tw_v, tb_v, sid_c, did_c, eid_c, ts_c, dt_c, big,
          gsem, ixsem0, ixsem1, stsem0, stsem1):
    sub = lax.axis_index("s")
    wid = sub * NC + lax.axis_index("c")
    t0 = wid * PER_TILE

    ixsem = (ixsem0, ixsem1)
    stsem = (stsem0, stsem1)

    pltpu.sync_copy(lu_hbm, lu_v)
    pltpu.sync_copy(tw_hbm, tw_v)
    pltpu.sync_copy(tb_hbm, tb_v)

    ws = tuple(tw_v[pl.ds(16 * j, 16)] for j in range(8))
    bs = tuple(tb_v[pl.ds(16 * j, 16)] for j in range(8))

    def stage(c, b, n):
        lo = t0 + c * CH
        pltpu.async_copy(src_hbm.at[pl.ds(lo, n)],
                         sid_c.at[b, pl.ds(0, n)], ixsem[b])
        pltpu.async_copy(dst_hbm.at[pl.ds(lo, n)],
                         did_c.at[b, pl.ds(0, n)], ixsem[b])
        pltpu.async_copy(idx_hbm.at[pl.ds(lo, n)],
                         eid_c.at[b, pl.ds(0, n)], ixsem[b])
        pltpu.async_copy(ts_hbm.at[pl.ds(lo, n)],
                         ts_c.at[b, pl.ds(0, n)], ixsem[b])

    def wait_stage(b, n):
        for ref in (sid_c, did_c, eid_c):
            pltpu.make_async_copy(src_hbm.at[pl.ds(0, n)],
                                  ref.at[b, pl.ds(0, n)], ixsem[b]).wait()
        pltpu.make_async_copy(ts_hbm.at[pl.ds(0, n)],
                              ts_c.at[b, pl.ds(0, n)], ixsem[b]).wait()

    def drain_store(b, n):
        pltpu.make_async_copy(big.at[b, pl.ds(0, n)],
                              out_hbm.at[pl.ds(0, n)], stsem[b]).wait()

    def process(c, b, n, stage_next):
        base = t0 + c * CH
        nb = (b + 1) % NB

        wait_stage(b, n)
        d1 = pltpu.async_copy(mem_hbm.at[sid_c.at[b, pl.ds(0, n)]],
                              big.at[b, pl.ds(0, n), pl.ds(0, D)], gsem)
        d2 = pltpu.async_copy(mem_hbm.at[did_c.at[b, pl.ds(0, n)]],
                              big.at[b, pl.ds(0, n), pl.ds(D, D)], gsem)
        d3 = pltpu.async_copy(feat_hbm.at[eid_c.at[b, pl.ds(0, n)]],
                              big.at[b, pl.ds(0, n), pl.ds(3 * D, D)], gsem)
        if stage_next:
            @pl.when(c + 1 < N_FULL)
            def _():
                stage(c + 1, nb, CH)

            @pl.when(c + 1 == N_FULL)
            def _():
                stage(N_FULL, nb, CH_T)

        for i in range(n // L):
            idx16 = sid_c[b, pl.ds(i * L, L)]
            lu16 = plsc.load_gather(lu_v, [idx16])
            dt_c[b, pl.ds(i * L, L)] = ts_c[b, pl.ds(i * L, L)] - lu16

        def ev(e, carry):
            eidx = jnp.zeros((L,), jnp.int32) + e
            d = plsc.load_gather(dt_c.at[b], [eidx])
            for j in range(8):
                t = d * ws[j] + bs[j]
                nn = (t + _BIG) - _BIG
                r = t - nn
                s = r * r
                p = jnp.float32(_COS_COEF[_NCOEF - 1])
                for k in range(_NCOEF - 2, -1, -1):
                    p = p * s + jnp.float32(_COS_COEF[k])
                big[b, e, pl.ds(2 * D + 16 * j, 16)] = p
            return carry
        lax.fori_loop(0, n, ev, 0)

        d1.wait()
        d2.wait()
        d3.wait()

        pltpu.async_copy(big.at[b, pl.ds(0, n)],
                         out_hbm.at[pl.ds(base, n)], stsem[b])

    stage(0, 0, CH)

    def pair(i, carry):
        for b in range(NB):
            @pl.when(i >= 1)
            def _(b=b):
                drain_store(b, CH)
            process(NB * i + b, b, CH, stage_next=True)
        return carry

    lax.fori_loop(0, N_FULL // NB, pair, 0)
    drain_store(0, CH)
    process(jnp.int32(N_FULL), 0, CH_T, stage_next=False)

    drain_store(1, CH)
    drain_store(0, CH_T)


_sc_call = pl.kernel(
    _body,
    out_type=jax.ShapeDtypeStruct((N_EVENTS, 4 * D), jnp.float32),
    mesh=plsc.VectorSubcoreMesh(core_axis_name="c", subcore_axis_name="s",
                                num_cores=NC, num_subcores=NS),
    compiler_params=pltpu.CompilerParams(needs_layout_passes=False),
    scratch_types=[
        pltpu.VMEM((N_NODES,), jnp.float32),
        pltpu.VMEM((D,), jnp.float32),
        pltpu.VMEM((D,), jnp.float32),
        pltpu.VMEM((NB, CH), jnp.int32),
        pltpu.VMEM((NB, CH), jnp.int32),
        pltpu.VMEM((NB, CH), jnp.int32),
        pltpu.VMEM((NB, CH), jnp.float32),
        pltpu.VMEM((NB, CH), jnp.float32),
        pltpu.VMEM((NB, CH, 4 * D), jnp.float32),
        pltpu.SemaphoreType.DMA,
        pltpu.SemaphoreType.DMA,
        pltpu.SemaphoreType.DMA,
        pltpu.SemaphoreType.DMA,
        pltpu.SemaphoreType.DMA,
    ],
)

def kernel(memory, last_update, src_nodes, dst_nodes, timestamps,
           event_features, indices, te_w, te_b):
    inv2pi = jnp.float32(1.0 / (2.0 * math.pi))
    return _sc_call(
        memory,
        last_update,
        src_nodes.astype(jnp.int32),
        dst_nodes.astype(jnp.int32),
        timestamps,
        event_features,
        indices.astype(jnp.int32),
        (te_w * inv2pi).astype(jnp.float32),
        (te_b * inv2pi).astype(jnp.float32),
    )
